# baseline (speedup 1.0000x reference)
import functools

import jax
import jax.numpy as jnp
from jax import lax
from jax.experimental import pallas as pl
from jax.experimental.pallas import tpu as pltpu
from jax.experimental.pallas import tpu_sc as plsc

MAX_SPAN = 255
Q = 2048
K = 2048
DEPTH = 64
EXT = 4096
NUM_WORKERS = 32
ROWS_PER_W = EXT // NUM_WORKERS
GATHER_W = 128
BQ = 8
NSLOT = 4


def _build_ext_sc(embeddings_padded, idx):
    mesh = plsc.VectorSubcoreMesh(core_axis_name="c", subcore_axis_name="s")

    @functools.partial(
        pl.kernel,
        mesh=mesh,
        out_type=jax.ShapeDtypeStruct((EXT, GATHER_W), jnp.float32),
        scratch_types=[
            pltpu.VMEM((ROWS_PER_W,), jnp.int32),
            pltpu.VMEM((ROWS_PER_W, GATHER_W), jnp.float32),
            pltpu.SemaphoreType.DMA,
        ],
    )
    def gather_kernel(table_hbm, idx_hbm, ext_hbm, idx_v, rows_v, sem):
        wid = lax.axis_index("s") * 2 + lax.axis_index("c")
        base = wid * ROWS_PER_W
        pltpu.sync_copy(idx_hbm.at[pl.ds(base, ROWS_PER_W)], idx_v)
        pltpu.async_copy(table_hbm.at[idx_v], rows_v, sem).wait()
        pltpu.sync_copy(rows_v, ext_hbm.at[pl.ds(base, ROWS_PER_W)])

    return gather_kernel(embeddings_padded, idx)


def _fanout_body(ext_ref, out_ref, buf, sems):
    g = pl.program_id(0)
    n = pl.num_programs(0)
    slot = lax.rem(g, NSLOT)
    q0 = g * BQ

    @pl.when(g >= NSLOT)
    def _():
        pltpu.make_async_copy(
            buf.at[slot], out_ref.at[pl.ds((g - NSLOT) * BQ, BQ)],
            sems.at[slot]).wait()

    for i in range(BQ):
        buf[slot, i] = ext_ref[pl.ds(K - 1 - (q0 + i), K), :DEPTH]
    pltpu.make_async_copy(buf.at[slot], out_ref.at[pl.ds(q0, BQ)],
                          sems.at[slot]).start()

    @pl.when(g == n - 1)
    def _():
        for s in range(NSLOT):
            src_g = n - NSLOT + s
            pltpu.make_async_copy(
                buf.at[lax.rem(src_g, NSLOT)],
                out_ref.at[pl.ds(src_g * BQ, BQ)],
                sems.at[lax.rem(src_g, NSLOT)]).wait()


def kernel(inputs, embeddings):
    del inputs
    idx = jnp.clip(jnp.arange(EXT, dtype=jnp.int32) - (K - 1) + (MAX_SPAN - 1),
                   0, 2 * MAX_SPAN - 2)
    emb_padded = jnp.pad(embeddings, ((0, 0), (0, GATHER_W - DEPTH)))
    ext = _build_ext_sc(emb_padded, idx)
    out = pl.pallas_call(
        _fanout_body,
        grid=(Q // BQ,),
        in_specs=[pl.BlockSpec((EXT, GATHER_W), lambda q: (0, 0))],
        out_specs=pl.BlockSpec(memory_space=pltpu.HBM),
        out_shape=jax.ShapeDtypeStruct((Q, K, DEPTH), jnp.float32),
        scratch_shapes=[
            pltpu.VMEM((NSLOT, BQ, K, DEPTH), jnp.float32),
            pltpu.SemaphoreType.DMA((NSLOT,)),
        ],
    )(ext)
    return out


# speedup vs baseline: 1.0340x; 1.0340x over previous
import functools

import jax
import jax.numpy as jnp
from jax import lax
from jax.experimental import pallas as pl
from jax.experimental.pallas import tpu as pltpu
from jax.experimental.pallas import tpu_sc as plsc

MAX_SPAN = 255
Q = 2048
K = 2048
DEPTH = 64
TROWS = 2 * MAX_SPAN - 1
NPACK = 2 * K
NUM_WORKERS = 32
ROWS_PER_W = NPACK // NUM_WORKERS
BQ = 8
NSLOT = 4


def _build_packed_sc(table_pairs, idx):
    mesh = plsc.VectorSubcoreMesh(core_axis_name="c", subcore_axis_name="s")

    @functools.partial(
        pl.kernel,
        mesh=mesh,
        out_type=jax.ShapeDtypeStruct((NPACK, 2 * DEPTH), jnp.float32),
        scratch_types=[
            pltpu.VMEM((ROWS_PER_W,), jnp.int32),
            pltpu.VMEM((ROWS_PER_W, 2 * DEPTH), jnp.float32),
            pltpu.SemaphoreType.DMA,
        ],
    )
    def gather_kernel(pairs_hbm, idx_hbm, packed_hbm, idx_v, rows_v, sem):
        wid = lax.axis_index("s") * 2 + lax.axis_index("c")
        base = wid * ROWS_PER_W
        pltpu.sync_copy(idx_hbm.at[pl.ds(base, ROWS_PER_W)], idx_v)
        pltpu.async_copy(pairs_hbm.at[idx_v], rows_v, sem).wait()
        pltpu.sync_copy(rows_v, packed_hbm.at[pl.ds(base, ROWS_PER_W)])

    return gather_kernel(table_pairs, idx)


def _fanout_body(p_ref, out_ref, buf, sems):
    g = pl.program_id(0)
    n = pl.num_programs(0)
    slot = lax.rem(g, NSLOT)
    q0 = g * BQ

    @pl.when(g >= NSLOT)
    def _():
        pltpu.make_async_copy(
            buf.at[slot], out_ref.at[pl.ds((g - NSLOT) * BQ, BQ)],
            sems.at[slot]).wait()

    for i in range(BQ):
        s = K - 1 - (q0 + i)
        start = lax.rem(s, 2) * K + lax.div(s, 2)
        buf[slot, i] = p_ref[pl.ds(start, K // 2), :]
    pltpu.make_async_copy(buf.at[slot], out_ref.at[pl.ds(q0, BQ)],
                          sems.at[slot]).start()

    @pl.when(g == n - 1)
    def _():
        for off in range(NSLOT):
            src_g = n - NSLOT + off
            pltpu.make_async_copy(
                buf.at[lax.rem(src_g, NSLOT)],
                out_ref.at[pl.ds(src_g * BQ, BQ)],
                sems.at[lax.rem(src_g, NSLOT)]).wait()


def kernel(inputs, embeddings):
    del inputs

    left = jnp.concatenate([embeddings[:1], embeddings], axis=0)
    right = jnp.concatenate([embeddings, embeddings[-1:]], axis=0)
    table_pairs = jnp.concatenate([left, right], axis=1)

    j2 = 2 * jnp.arange(K, dtype=jnp.int32)
    idx_e = jnp.clip(j2 - (K - 1 - (MAX_SPAN - 1)), -1, TROWS - 1) + 1
    idx_o = jnp.clip(j2 - (K - 2 - (MAX_SPAN - 1)), -1, TROWS - 1) + 1
    idx = jnp.concatenate([idx_e, idx_o])

    packed = _build_packed_sc(table_pairs, idx)

    out128 = pl.pallas_call(
        _fanout_body,
        grid=(Q // BQ,),
        in_specs=[pl.BlockSpec((NPACK, 2 * DEPTH), lambda q: (0, 0))],
        out_specs=pl.BlockSpec(memory_space=pltpu.HBM),
        out_shape=jax.ShapeDtypeStruct((Q, K // 2, 2 * DEPTH), jnp.float32),
        scratch_shapes=[
            pltpu.VMEM((NSLOT, BQ, K // 2, 2 * DEPTH), jnp.float32),
            pltpu.SemaphoreType.DMA((NSLOT,)),
        ],
    )(packed)
    return out128.reshape(Q, K, DEPTH)


# speedup vs baseline: 3.9752x; 3.8444x over previous
import functools

import jax
import jax.numpy as jnp
from jax import lax
from jax.experimental import pallas as pl
from jax.experimental.pallas import tpu as pltpu
from jax.experimental.pallas import tpu_sc as plsc

MAX_SPAN = 255
Q = 2048
K = 2048
DEPTH = 64
TROWS = 2 * MAX_SPAN - 1
EXT = 4096
NUM_WORKERS = 32
ROWS_PER_W = EXT // NUM_WORKERS
GATHER_W = 128
BQ = 8


def _build_ext_sc(embeddings_padded, idx):
    mesh = plsc.VectorSubcoreMesh(core_axis_name="c", subcore_axis_name="s")

    @functools.partial(
        pl.kernel,
        mesh=mesh,
        out_type=jax.ShapeDtypeStruct((EXT, GATHER_W), jnp.float32),
        scratch_types=[
            pltpu.VMEM((ROWS_PER_W,), jnp.int32),
            pltpu.VMEM((ROWS_PER_W, GATHER_W), jnp.float32),
            pltpu.SemaphoreType.DMA,
        ],
    )
    def gather_kernel(table_hbm, idx_hbm, ext_hbm, idx_v, rows_v, sem):
        wid = lax.axis_index("s") * 2 + lax.axis_index("c")
        base = wid * ROWS_PER_W
        pltpu.sync_copy(idx_hbm.at[pl.ds(base, ROWS_PER_W)], idx_v)
        pltpu.async_copy(table_hbm.at[idx_v], rows_v, sem).wait()
        pltpu.sync_copy(rows_v, ext_hbm.at[pl.ds(base, ROWS_PER_W)])

    return gather_kernel(embeddings_padded, idx)


WIN = K + 128


def _fanout_body(extt_ref, out_ref):
    q0 = pl.program_id(0) * BQ
    for i in range(BQ):
        s = K - 1 - (q0 + i)
        r = lax.rem(s, 128)
        base = pl.multiple_of(s - r, 128)
        win = extt_ref[:, pl.ds(base, WIN)]
        rolled = pltpu.roll(win, WIN - r, axis=1)
        out_ref[i] = rolled[:, :K]


def kernel(inputs, embeddings):
    del inputs

    idx = jnp.clip(jnp.arange(EXT, dtype=jnp.int32) - (K - 1) + (MAX_SPAN - 1),
                   0, TROWS - 1)
    emb_padded = jnp.pad(embeddings, ((0, 0), (0, GATHER_W - DEPTH)))
    ext = _build_ext_sc(emb_padded, idx)
    extt = ext[:, :DEPTH].T

    out_t = pl.pallas_call(
        _fanout_body,
        grid=(Q // BQ,),
        in_specs=[pl.BlockSpec((DEPTH, EXT), lambda q: (0, 0))],
        out_specs=pl.BlockSpec((BQ, DEPTH, K), lambda q: (q, 0, 0)),
        out_shape=jax.ShapeDtypeStruct((Q, DEPTH, K), jnp.float32),
    )(extt)
    return jnp.swapaxes(out_t, 1, 2)


# speedup vs baseline: 3.9972x; 1.0055x over previous
import functools

import jax
import jax.numpy as jnp
from jax import lax
from jax.experimental import pallas as pl
from jax.experimental.pallas import tpu as pltpu
from jax.experimental.pallas import tpu_sc as plsc

MAX_SPAN = 255
Q = 2048
K = 2048
DEPTH = 64
TROWS = 2 * MAX_SPAN - 1
EXT = 4096
NUM_WORKERS = 16
ROWS_PER_W = EXT // NUM_WORKERS
GATHER_W = 128
BQ = 8


def _build_ext_sc(embeddings_padded):
    mesh = plsc.VectorSubcoreMesh(core_axis_name="c", subcore_axis_name="s",
                                  num_cores=1)

    @functools.partial(
        pl.kernel,
        mesh=mesh,
        out_type=jax.ShapeDtypeStruct((EXT, GATHER_W), jnp.float32),
        scratch_types=[
            pltpu.VMEM((2, 128), jnp.int32),
            pltpu.VMEM((ROWS_PER_W, GATHER_W), jnp.float32),
            pltpu.SemaphoreType.DMA,
        ],
    )
    def gather_kernel(table_hbm, ext_hbm, idx_v, rows_v, sem):
        base = lax.axis_index("s") * ROWS_PER_W
        for c in range(ROWS_PER_W // 16):
            lane = lax.iota(jnp.int32, 16)
            m = base + c * 16 + lane
            val = jnp.clip(m - (K - 1) + (MAX_SPAN - 1), 0, TROWS - 1)
            idx_v[c // 8, pl.ds((c % 8) * 16, 16)] = val
        cp0 = pltpu.async_copy(table_hbm.at[idx_v.at[0]],
                               rows_v.at[pl.ds(0, 128)], sem)
        cp1 = pltpu.async_copy(table_hbm.at[idx_v.at[1]],
                               rows_v.at[pl.ds(128, 128)], sem)
        cp0.wait()
        cp1.wait()
        pltpu.sync_copy(rows_v, ext_hbm.at[pl.ds(base, ROWS_PER_W)])

    return gather_kernel(embeddings_padded)


WIN = K + 128


def _fanout_body(extt_ref, out_ref):
    q0 = pl.program_id(0) * BQ
    for i in range(BQ):
        s = K - 1 - (q0 + i)
        r = lax.rem(s, 128)
        base = pl.multiple_of(s - r, 128)
        win = extt_ref[:, pl.ds(base, WIN)]
        rolled = pltpu.roll(win, WIN - r, axis=1)
        out_ref[i] = rolled[:, :K]


def kernel(inputs, embeddings):
    del inputs

    emb_padded = jnp.pad(embeddings, ((0, 0), (0, GATHER_W - DEPTH)))
    ext = _build_ext_sc(emb_padded)
    extt = ext[:, :DEPTH].T

    out_t = pl.pallas_call(
        _fanout_body,
        grid=(Q // BQ,),
        in_specs=[pl.BlockSpec((DEPTH, EXT), lambda q: (0, 0))],
        out_specs=pl.BlockSpec((BQ, DEPTH, K), lambda q: (q, 0, 0)),
        out_shape=jax.ShapeDtypeStruct((Q, DEPTH, K), jnp.float32),
    )(extt)
    return jnp.swapaxes(out_t, 1, 2)


# speedup vs baseline: 4.3729x; 1.0940x over previous
import functools

import jax
import jax.numpy as jnp
from jax import lax
from jax.experimental import pallas as pl
from jax.experimental.pallas import tpu as pltpu
from jax.experimental.pallas import tpu_sc as plsc

MAX_SPAN = 255
Q = 2048
K = 2048
DEPTH = 64
TROWS = 2 * MAX_SPAN - 1
EXT = 4096
NUM_WORKERS = 16
ROWS_PER_W = EXT // NUM_WORKERS
GATHER_W = 128
BQ = 16


def _build_ext_sc(embeddings_padded):
    mesh = plsc.VectorSubcoreMesh(core_axis_name="c", subcore_axis_name="s",
                                  num_cores=1)

    @functools.partial(
        pl.kernel,
        mesh=mesh,
        out_type=jax.ShapeDtypeStruct((EXT, GATHER_W), jnp.float32),
        scratch_types=[
            pltpu.VMEM((2, 128), jnp.int32),
            pltpu.VMEM((ROWS_PER_W, GATHER_W), jnp.float32),
            pltpu.SemaphoreType.DMA,
        ],
    )
    def gather_kernel(table_hbm, ext_hbm, idx_v, rows_v, sem):
        base = lax.axis_index("s") * ROWS_PER_W
        for c in range(ROWS_PER_W // 16):
            lane = lax.iota(jnp.int32, 16)
            m = base + c * 16 + lane
            val = jnp.clip(m - (K - 1) + (MAX_SPAN - 1), 0, TROWS - 1)
            idx_v[c // 8, pl.ds((c % 8) * 16, 16)] = val
        cp0 = pltpu.async_copy(table_hbm.at[idx_v.at[0]],
                               rows_v.at[pl.ds(0, 128)], sem)
        cp1 = pltpu.async_copy(table_hbm.at[idx_v.at[1]],
                               rows_v.at[pl.ds(128, 128)], sem)
        cp0.wait()
        cp1.wait()
        pltpu.sync_copy(rows_v, ext_hbm.at[pl.ds(base, ROWS_PER_W)])

    return gather_kernel(embeddings_padded)


WIN = K + 128


def _fanout_body(extt_ref, out_ref):
    q0 = pl.program_id(0) * BQ
    for i in range(BQ):
        s = K - 1 - (q0 + i)
        r = lax.rem(s, 128)
        base = pl.multiple_of(s - r, 128)
        win = extt_ref[:, pl.ds(base, WIN)]
        rolled = pltpu.roll(win, WIN - r, axis=1)
        out_ref[i] = rolled[:, :K]


def kernel(inputs, embeddings):
    del inputs

    emb_padded = jnp.pad(embeddings, ((0, 0), (0, GATHER_W - DEPTH)))
    ext = _build_ext_sc(emb_padded)
    extt = ext[:, :DEPTH].T

    out_t = pl.pallas_call(
        _fanout_body,
        grid=(Q // BQ,),
        in_specs=[pl.BlockSpec((DEPTH, EXT), lambda q: (0, 0))],
        out_specs=pl.BlockSpec((BQ, DEPTH, K), lambda q: (q, 0, 0)),
        out_shape=jax.ShapeDtypeStruct((Q, DEPTH, K), jnp.float32),
    )(extt)
    return jnp.swapaxes(out_t, 1, 2)


# speedup vs baseline: 4.5079x; 1.0309x over previous
import functools

import jax
import jax.numpy as jnp
from jax import lax
from jax.experimental import pallas as pl
from jax.experimental.pallas import tpu as pltpu
from jax.experimental.pallas import tpu_sc as plsc

MAX_SPAN = 255
Q = 2048
K = 2048
DEPTH = 64
TROWS = 2 * MAX_SPAN - 1
EXT = 4096
NUM_WORKERS = 16
ROWS_PER_W = EXT // NUM_WORKERS
GATHER_W = 128
BQ = 32


def _build_ext_sc(embeddings_padded):
    mesh = plsc.VectorSubcoreMesh(core_axis_name="c", subcore_axis_name="s",
                                  num_cores=1)

    @functools.partial(
        pl.kernel,
        mesh=mesh,
        out_type=jax.ShapeDtypeStruct((EXT, GATHER_W), jnp.float32),
        scratch_types=[
            pltpu.VMEM((2, 128), jnp.int32),
            pltpu.VMEM((ROWS_PER_W, GATHER_W), jnp.float32),
            pltpu.SemaphoreType.DMA,
        ],
    )
    def gather_kernel(table_hbm, ext_hbm, idx_v, rows_v, sem):
        base = lax.axis_index("s") * ROWS_PER_W
        for c in range(ROWS_PER_W // 16):
            lane = lax.iota(jnp.int32, 16)
            m = base + c * 16 + lane
            val = jnp.clip(m - (K - 1) + (MAX_SPAN - 1), 0, TROWS - 1)
            idx_v[c // 8, pl.ds((c % 8) * 16, 16)] = val
        cp0 = pltpu.async_copy(table_hbm.at[idx_v.at[0]],
                               rows_v.at[pl.ds(0, 128)], sem)
        cp1 = pltpu.async_copy(table_hbm.at[idx_v.at[1]],
                               rows_v.at[pl.ds(128, 128)], sem)
        cp0.wait()
        cp1.wait()
        pltpu.sync_copy(rows_v, ext_hbm.at[pl.ds(base, ROWS_PER_W)])

    return gather_kernel(embeddings_padded)


WIN = K + 128


def _fanout_body(extt_ref, out_ref):
    q0 = pl.program_id(0) * BQ
    for i in range(BQ):
        s = K - 1 - (q0 + i)
        r = lax.rem(s, 128)
        base = pl.multiple_of(s - r, 128)
        win = extt_ref[:, pl.ds(base, WIN)]
        rolled = pltpu.roll(win, WIN - r, axis=1)
        out_ref[i] = rolled[:, :K]


def kernel(inputs, embeddings):
    del inputs

    emb_padded = jnp.pad(embeddings, ((0, 0), (0, GATHER_W - DEPTH)))
    ext = _build_ext_sc(emb_padded)
    extt = ext[:, :DEPTH].T

    out_t = pl.pallas_call(
        _fanout_body,
        grid=(Q // BQ,),
        in_specs=[pl.BlockSpec((DEPTH, EXT), lambda q: (0, 0))],
        out_specs=pl.BlockSpec((BQ, DEPTH, K), lambda q: (q, 0, 0)),
        out_shape=jax.ShapeDtypeStruct((Q, DEPTH, K), jnp.float32),
    )(extt)
    return jnp.swapaxes(out_t, 1, 2)
